# Initial kernel scaffold; baseline (speedup 1.0000x reference)
import jax
import jax.numpy as jnp
from jax.experimental import pallas as pl


def kernel(embeddings, labels):
    raise NotImplementedError("write your pallas kernel here")


# speedup vs baseline: 12.8974x; 12.8974x over previous
import jax
import jax.numpy as jnp
from jax import lax
from jax.experimental import pallas as pl

_BETA = 0.3
_C = 16


def _body(e_ref, lab_ref, out_ref):
    E = e_ref[...]
    lab = lab_ref[...]
    classes = lax.broadcasted_iota(jnp.int32, (1, _C), 1)
    M = (lab == classes).astype(jnp.float32)

    f32 = jnp.float32
    dn_t = (((0,), (0,)), ((), ()))
    dn_n = (((1,), (0,)), ((), ()))

    cnt = jnp.sum(M, axis=0, keepdims=True)
    S = lax.dot_general(M, E, dn_t, preferred_element_type=f32)
    sq = jnp.sum(E * E, axis=1, keepdims=True)
    q = lax.dot_general(M, sq, dn_t, preferred_element_type=f32)

    safe = jnp.maximum(cnt, 1.0)
    mu = S / safe.reshape(_C, 1)
    n2 = jnp.sum(mu * mu, axis=1, keepdims=True)
    mean_d = q / safe.reshape(_C, 1) - n2

    D = lax.dot_general(E, mu, (((1,), (1,)), ((), ())),
                        preferred_element_type=f32)
    dot_i = jnp.sum(D * M, axis=1, keepdims=True)
    off = lax.dot_general(M, n2 - mean_d, dn_n,
                          preferred_element_type=f32)
    t = sq - 2.0 * dot_i + off
    T2 = lax.dot_general(M, t * t, dn_t, preferred_element_type=f32)

    cnt_c = cnt.reshape(_C, 1)
    var = T2 / jnp.maximum(cnt_c - 1.0, 1.0)
    total = jnp.sum(jnp.where(cnt_c > 1.0, var, 0.0))
    nu = jnp.sum((cnt > 0.0).astype(f32))
    loss = _BETA * total / jnp.maximum(nu, 1.0)
    out_ref[...] = jnp.full((1, 1), loss, dtype=f32)


def kernel(embeddings, labels):
    lab = labels.astype(jnp.int32).reshape(-1, 1)
    out = pl.pallas_call(
        _body,
        out_shape=jax.ShapeDtypeStruct((1, 1), jnp.float32),
    )(embeddings, lab)
    return out[0, 0]
